# baseline (speedup 1.0000x reference)
import functools

import jax
import jax.numpy as jnp
from jax.experimental import pallas as pl


_BLOCK_ROWS = 512


def _body(x_ref, mins_ref, flag_ref, *, last_dims_equal):
    mins_ref[...] = jnp.min(x_ref[...], axis=1)

    @pl.when(pl.program_id(0) == 0)
    def _():
        flag_ref[...] = jnp.full((1, 1), 1.0 if last_dims_equal else 0.0,
                                 jnp.float32)


def kernel(x):
    b0, b1, k = x.shape
    rows = b0 * b1
    xr = x.reshape(rows, k)
    body = functools.partial(_body, last_dims_equal=(b1 == k))
    mins, flag = pl.pallas_call(
        body,
        grid=(rows // _BLOCK_ROWS,),
        in_specs=[pl.BlockSpec((_BLOCK_ROWS, k), lambda i: (i, 0))],
        out_specs=[
            pl.BlockSpec((_BLOCK_ROWS,), lambda i: (i,)),
            pl.BlockSpec((1, 1), lambda i: (0, 0)),
        ],
        out_shape=[
            jax.ShapeDtypeStruct((rows,), jnp.float32),
            jax.ShapeDtypeStruct((1, 1), jnp.float32),
        ],
    )(xr)
    del mins
    return flag[0, 0].astype(jnp.bool_)
